# baseline (speedup 1.0000x reference)
import functools

import jax
import jax.numpy as jnp
from jax import lax
from jax.experimental import pallas as pl
from jax.experimental.pallas import tpu as pltpu
from jax.experimental.pallas import tpu_sc as plsc

NC = 2
NS = 16
L = 16
NW = NC * NS
NBUF = 3
C = 16


def kernel(global_info, x, ptr):
    B, D = global_info.shape
    TOTAL = x.shape[0]
    rows_per_w = TOTAL // NW
    nchunks = rows_per_w // C

    mesh = plsc.VectorSubcoreMesh(core_axis_name="c", subcore_axis_name="s")

    @functools.partial(
        pl.kernel,
        out_type=jax.ShapeDtypeStruct((TOTAL, 2 * D), jnp.float32),
        mesh=mesh,
        scratch_types=[
            pltpu.VMEM((L,), jnp.int32),
            [pltpu.VMEM((C, 2 * D), jnp.float32) for _ in range(NBUF)],
            [pltpu.SemaphoreType.DMA for _ in range(NBUF)],
            [pltpu.SemaphoreType.DMA for _ in range(NBUF)],
            [pltpu.SemaphoreType.DMA for _ in range(NBUF)],
        ],
    )
    def run(g_hbm, x_hbm, ptr_hbm, out_hbm, ptr_v, bufs, si, sg, so):
        wid = lax.axis_index("s") * NC + lax.axis_index("c")
        base = wid * rows_per_w

        pltpu.sync_copy(ptr_hbm.at[pl.ds(0, L)], ptr_v)
        pv = ptr_v[...]
        ones = jnp.full((L,), 1, jnp.int32)
        zeros = jnp.zeros((L,), jnp.int32)
        pbs = [
            pv.at[jnp.full((L,), b, jnp.int32)].get(mode="promise_in_bounds")
            for b in range(1, B)
        ]

        def seg_of(row0):
            rows = row0 + lax.iota(jnp.int32, L)
            seg = zeros
            for pb in pbs:
                seg = seg + jnp.where(pb <= rows, ones, zeros)
            return seg

        def start_in(k, j):
            row0 = base + k * C
            pltpu.async_copy(x_hbm.at[pl.ds(row0, C)],
                             bufs[j].at[:, pl.ds(0, D)], si[j])

        def wait_in(j):
            pltpu.make_async_copy(x_hbm.at[pl.ds(0, C)],
                                  bufs[j].at[:, pl.ds(0, D)], si[j]).wait()

        def start_out(k, j):
            row0 = base + k * C
            pltpu.async_copy(bufs[j], out_hbm.at[pl.ds(row0, C)], so[j])

        def wait_out(j):
            pltpu.make_async_copy(bufs[j], out_hbm.at[pl.ds(0, C)],
                                  so[j]).wait()

        def wait_gather(j):
            pltpu.make_async_copy(x_hbm.at[pl.ds(0, C)],
                                  bufs[j].at[:, pl.ds(D, D)], sg[j]).wait()

        def issue_gather_if_needed(k, j, cur_j, gate):
            seg = seg_of(base + k * C)
            s0 = seg[0]
            s1 = seg[L - 1]
            need = jnp.logical_and(
                gate, jnp.logical_or(s0 != s1, cur_j != s0))

            @pl.when(need)
            def _():
                pltpu.async_copy(g_hbm.at[seg],
                                 bufs[j].at[:, pl.ds(D, D)], sg[j])

            return jnp.where(s0 == s1, s0, jnp.int32(-1)), need

        true = jnp.bool_(True)
        start_in(0, 0)
        cur0, pend0 = issue_gather_if_needed(0, 0, jnp.int32(-1), true)
        start_in(1, 1)
        cur1, pend1 = issue_gather_if_needed(1, 1, jnp.int32(-1), true)

        def step(kb, carry):
            cur = list(carry[:NBUF])
            pend = list(carry[NBUF:])
            for j in range(NBUF):
                j2 = (j + 2) % NBUF
                k = kb * NBUF + j
                wait_in(j)

                @pl.when(pend[j])
                def _():
                    wait_gather(j)

                start_out(k, j)

                @pl.when(k >= 1)
                def _():
                    wait_out(j2)

                @pl.when(k + 2 < nchunks)
                def _():
                    start_in(k + 2, j2)

                inrange = k + 2 < nchunks
                cur_new, pend_new = issue_gather_if_needed(
                    k + 2, j2, cur[j2], inrange)
                cur[j2] = jnp.where(inrange, cur_new, cur[j2])
                pend[j2] = pend_new
            return tuple(cur) + tuple(pend)

        carry = lax.fori_loop(
            0, (nchunks - 1) // NBUF, step,
            (cur0, cur1, jnp.int32(-1), pend0, pend1, jnp.bool_(False)))

        k = nchunks - 1
        wait_in(0)

        @pl.when(carry[NBUF])
        def _():
            wait_gather(0)

        start_out(k, 0)
        wait_out((nchunks - 2) % NBUF)
        wait_out((nchunks - 1) % NBUF)

    return run(global_info, x, ptr)


# speedup vs baseline: 1.0156x; 1.0156x over previous
import functools

import jax
import jax.numpy as jnp
from jax import lax
from jax.experimental import pallas as pl
from jax.experimental.pallas import tpu as pltpu
from jax.experimental.pallas import tpu_sc as plsc

NC = 2
NS = 16
L = 16
NW = NC * NS
NBUF = 2
C = 16


def kernel(global_info, x, ptr):
    B, D = global_info.shape
    TOTAL = x.shape[0]
    rows_per_w = TOTAL // NW
    nchunks = rows_per_w // C

    mesh = plsc.VectorSubcoreMesh(core_axis_name="c", subcore_axis_name="s")

    @functools.partial(
        pl.kernel,
        out_type=jax.ShapeDtypeStruct((TOTAL, 2 * D), jnp.float32),
        mesh=mesh,
        scratch_types=[
            pltpu.VMEM((L,), jnp.int32),
            [pltpu.VMEM((C, 2 * D), jnp.float32) for _ in range(NBUF)],
            [pltpu.SemaphoreType.DMA for _ in range(NBUF)],
            [pltpu.SemaphoreType.DMA for _ in range(NBUF)],
            [pltpu.SemaphoreType.DMA for _ in range(NBUF)],
        ],
    )
    def run(g_hbm, x_hbm, ptr_hbm, out_hbm, ptr_v, bufs, si, sg, so):
        wid = lax.axis_index("s") * NC + lax.axis_index("c")
        base = wid * rows_per_w

        pltpu.sync_copy(ptr_hbm.at[pl.ds(0, L)], ptr_v)
        pv = ptr_v[...]
        ones = jnp.full((L,), 1, jnp.int32)
        zeros = jnp.zeros((L,), jnp.int32)
        pbs = [
            pv.at[jnp.full((L,), b, jnp.int32)].get(mode="promise_in_bounds")
            for b in range(1, B)
        ]
        pscal = [pv[b] for b in range(1, B)]

        def seg_of(row0):
            rows = row0 + lax.iota(jnp.int32, L)
            seg = zeros
            for pb in pbs:
                seg = seg + jnp.where(pb <= rows, ones, zeros)
            return seg

        def start_in(k, j):
            row0 = base + k * C
            pltpu.async_copy(x_hbm.at[pl.ds(row0, C)],
                             bufs[j].at[:, pl.ds(0, D)], si[j])

        def wait_in(j):
            pltpu.make_async_copy(x_hbm.at[pl.ds(0, C)],
                                  bufs[j].at[:, pl.ds(0, D)], si[j]).wait()

        def start_out(k, j):
            row0 = base + k * C
            pltpu.async_copy(bufs[j], out_hbm.at[pl.ds(row0, C)], so[j])

        def wait_out(j):
            pltpu.make_async_copy(bufs[j], out_hbm.at[pl.ds(0, C)],
                                  so[j]).wait()

        def wait_gather(j):
            pltpu.make_async_copy(x_hbm.at[pl.ds(0, C)],
                                  bufs[j].at[:, pl.ds(D, D)], sg[j]).wait()

        def issue_gather_if_needed(k, j, cur_j, gate):
            row0 = base + k * C
            s0 = jnp.int32(0)
            s1 = jnp.int32(0)
            one = jnp.int32(1)
            zero = jnp.int32(0)
            for pb in pscal:
                s0 = s0 + jnp.where(pb <= row0, one, zero)
                s1 = s1 + jnp.where(pb <= row0 + (C - 1), one, zero)
            need = jnp.logical_and(
                gate, jnp.logical_or(s0 != s1, cur_j != s0))

            @pl.when(need)
            def _():
                pltpu.async_copy(g_hbm.at[seg_of(row0)],
                                 bufs[j].at[:, pl.ds(D, D)], sg[j])

            return jnp.where(s0 == s1, s0, jnp.int32(-1)), need

        start_in(0, 0)
        cur0, pend0 = issue_gather_if_needed(0, 0, jnp.int32(-1),
                                             jnp.bool_(True))

        def step(kb, carry):
            cur = [carry[0], carry[1]]
            pend = [carry[2], carry[3]]
            for j in range(NBUF):
                jn = (j + 1) % NBUF
                k = kb * NBUF + j
                wait_in(j)

                @pl.when(pend[j])
                def _():
                    wait_gather(j)

                start_out(k, j)

                @pl.when(k >= 1)
                def _():
                    wait_out(jn)

                @pl.when(k + 1 < nchunks)
                def _():
                    start_in(k + 1, jn)

                inrange = k + 1 < nchunks
                cur_new, pend_new = issue_gather_if_needed(
                    k + 1, jn, cur[jn], inrange)
                cur[jn] = jnp.where(inrange, cur_new, cur[jn])
                pend[jn] = pend_new
            return (cur[0], cur[1], pend[0], pend[1])

        lax.fori_loop(0, nchunks // NBUF, step,
                      (cur0, jnp.int32(-1), pend0, jnp.bool_(False)))
        wait_out((nchunks - 1) % NBUF)

    return run(global_info, x, ptr)
